# Initial kernel scaffold; baseline (speedup 1.0000x reference)
import jax
import jax.numpy as jnp
from jax.experimental import pallas as pl


def kernel(inputs, W_enc, b_enc, codebook, W_dec, b_dec):
    raise NotImplementedError("write your pallas kernel here")


# speedup vs baseline: 1.0910x; 1.0910x over previous
import functools

import jax
import jax.numpy as jnp
from jax import lax
from jax.experimental import pallas as pl
from jax.experimental.pallas import tpu as pltpu
from jax.experimental.pallas import tpu_sc as plsc

P = 16
D = 64
DP = 128
K = 1024
TOK_BLK = 768

_NC = 2
_NS = 16
_NW = _NC * _NS
_TOK_TOTAL = 9216
_BPW = _TOK_TOTAL // _NW
_CHUNK = 96
_NCHUNK = _BPW // _CHUNK


def _encode_argmin_body(patches_ref, w_ref, b_ref, cb_ref, cn_ref, z_ref, idx_ref):
    z = jnp.dot(patches_ref[...], w_ref[...],
                preferred_element_type=jnp.float32) + b_ref[...]
    z_ref[...] = z
    d = cn_ref[...] - lax.dot_general(
        z, cb_ref[...], (((1,), (1,)), ((), ())),
        preferred_element_type=jnp.float32)
    dmin = jnp.min(d, axis=1, keepdims=True)
    iota = lax.broadcasted_iota(jnp.int32, d.shape, 1)
    idx = jnp.min(jnp.where(d == dmin, iota, K), axis=1)
    idx_ref[...] = idx[None, None, :]


def _decode_loss_body(patches_ref, z_ref, emb_ref, wd_ref, bd_ref,
                      recon_ref, vq_ref, mse_ref):
    emb = emb_ref[...]
    recon = jnp.dot(emb, wd_ref[...],
                    preferred_element_type=jnp.float32) + bd_ref[...]
    recon_ref[...] = recon
    dz = z_ref[...] - emb
    vq_p = jnp.sum(jnp.sum(dz * dz, axis=1, keepdims=True), axis=0,
                   keepdims=True)
    dr = recon - patches_ref[...]
    mse_p = jnp.sum(jnp.sum(dr * dr, axis=1, keepdims=True), axis=0,
                    keepdims=True)

    @pl.when(pl.program_id(0) == 0)
    def _init():
        vq_ref[...] = vq_p
        mse_ref[...] = mse_p

    @pl.when(pl.program_id(0) != 0)
    def _acc():
        vq_ref[...] += vq_p
        mse_ref[...] += mse_p


@functools.partial(
    pl.kernel,
    mesh=plsc.VectorSubcoreMesh(core_axis_name="c", subcore_axis_name="s"),
    out_type=jax.ShapeDtypeStruct((_TOK_TOTAL, DP), jnp.float32),
    scratch_types=[
        pltpu.VMEM((_NCHUNK, _CHUNK), jnp.int32),
        pltpu.VMEM((_BPW, DP), jnp.float32),
        pltpu.SemaphoreType.DMA,
    ],
)
def _sc_gather(cb_hbm, idx_hbm, out_hbm, idx_v, rows_v, sem):
    wid = lax.axis_index("s") * _NC + lax.axis_index("c")
    base = wid * _BPW
    pltpu.sync_copy(idx_hbm.at[wid], idx_v)
    for j in range(_NCHUNK):
        pltpu.async_copy(cb_hbm.at[idx_v.at[j]],
                         rows_v.at[pl.ds(j * _CHUNK, _CHUNK)], sem).wait()
    pltpu.sync_copy(rows_v, out_hbm.at[pl.ds(base, _BPW)])


def kernel(inputs, W_enc, b_enc, codebook, W_dec, b_dec):
    Bb, Cc, H, W = inputs.shape
    h, w = H // P, W // P
    T = Bb * h * w
    pd = Cc * P * P
    x = inputs.reshape(Bb, Cc, h, P, w, P).transpose(0, 2, 4, 1, 3, 5)
    x = x.reshape(T, pd)
    cn_half = 0.5 * jnp.sum(codebook * codebook, axis=1)[None, :]
    pad = DP - D
    W_enc_p = jnp.pad(W_enc, ((0, 0), (0, pad)))
    b_enc_p = jnp.pad(b_enc.reshape(1, D), ((0, 0), (0, pad)))
    cb_p = jnp.pad(codebook, ((0, 0), (0, pad)))
    W_dec_p = jnp.pad(W_dec, ((0, pad), (0, 0)))
    nblk = T // TOK_BLK

    z, idx3 = pl.pallas_call(
        _encode_argmin_body,
        grid=(nblk,),
        in_specs=[
            pl.BlockSpec((TOK_BLK, pd), lambda i: (i, 0)),
            pl.BlockSpec((pd, DP), lambda i: (0, 0)),
            pl.BlockSpec((1, DP), lambda i: (0, 0)),
            pl.BlockSpec((K, DP), lambda i: (0, 0)),
            pl.BlockSpec((1, K), lambda i: (0, 0)),
        ],
        out_specs=[
            pl.BlockSpec((TOK_BLK, DP), lambda i: (i, 0)),
            pl.BlockSpec((1, 1, TOK_BLK), lambda i: (i, 0, 0)),
        ],
        out_shape=[
            jax.ShapeDtypeStruct((T, DP), jnp.float32),
            jax.ShapeDtypeStruct((nblk, 1, TOK_BLK), jnp.int32),
        ],
    )(x, W_enc_p, b_enc_p, cb_p, cn_half)

    idx = idx3.reshape(_NW, _NCHUNK, _CHUNK)
    emb = _sc_gather(cb_p, idx)

    recon_p, vq_sse, mse_sse = pl.pallas_call(
        _decode_loss_body,
        grid=(nblk,),
        in_specs=[
            pl.BlockSpec((TOK_BLK, pd), lambda i: (i, 0)),
            pl.BlockSpec((TOK_BLK, DP), lambda i: (i, 0)),
            pl.BlockSpec((TOK_BLK, DP), lambda i: (i, 0)),
            pl.BlockSpec((DP, pd), lambda i: (0, 0)),
            pl.BlockSpec((1, pd), lambda i: (0, 0)),
        ],
        out_specs=[
            pl.BlockSpec((TOK_BLK, pd), lambda i: (i, 0)),
            pl.BlockSpec((1, 1), lambda i: (0, 0)),
            pl.BlockSpec((1, 1), lambda i: (0, 0)),
        ],
        out_shape=[
            jax.ShapeDtypeStruct((T, pd), jnp.float32),
            jax.ShapeDtypeStruct((1, 1), jnp.float32),
            jax.ShapeDtypeStruct((1, 1), jnp.float32),
        ],
    )(x, z, emb, W_dec_p, b_dec.reshape(1, pd))

    recon = recon_p.reshape(Bb, h, w, Cc, P, P).transpose(0, 3, 1, 4, 2, 5)
    recon = recon.reshape(Bb, Cc, H, W)
    mse = mse_sse[0, 0] / (Bb * Cc * H * W)
    vq = vq_sse[0, 0] / (T * D)
    loss = 1.25 * vq + mse
    return (loss, mse, recon)


# speedup vs baseline: 1.9868x; 1.8211x over previous
import functools

import jax
import jax.numpy as jnp
from jax import lax
from jax.experimental import pallas as pl
from jax.experimental.pallas import tpu as pltpu
from jax.experimental.pallas import tpu_sc as plsc

P = 16
D = 64
DP = 128
K = 1024
TOK_BLK = 768

_NC = 2
_NS = 16
_NW = _NC * _NS
_TOK_TOTAL = 9216
_BPW = _TOK_TOTAL // _NW
_CHUNK = 96
_NCHUNK = _BPW // _CHUNK


def _patchify_block(x):
    t = x.reshape(3, 24, P, 24, P).transpose(1, 3, 0, 2, 4)
    return t.reshape(576, 3 * P * P)


def _unpatchify_block(t):
    x = t.reshape(24, 24, 3, P, P).transpose(2, 0, 3, 1, 4)
    return x.reshape(3, 384, 384)


def _encode_argmin_body(x_ref, w_ref, b_ref, cb_ref, cn_ref, z_ref, idx_ref):
    patches = _patchify_block(x_ref[0])
    z = jnp.dot(patches, w_ref[...],
                preferred_element_type=jnp.float32) + b_ref[...]
    z_ref[...] = z
    d = cn_ref[...] - lax.dot_general(
        z, cb_ref[...], (((1,), (1,)), ((), ())),
        preferred_element_type=jnp.float32)
    dmin = jnp.min(d, axis=1, keepdims=True)
    iota = lax.broadcasted_iota(jnp.int32, d.shape, 1)
    idx = jnp.min(jnp.where(d == dmin, iota, K), axis=1)
    idx_ref[...] = idx[None, None, :]


def _decode_loss_body(x_ref, z_ref, emb_ref, wd_ref, bd_ref,
                      recon_ref, vq_ref, mse_ref):
    emb = emb_ref[...]
    recon = jnp.dot(emb, wd_ref[...],
                    preferred_element_type=jnp.float32) + bd_ref[...]
    recon_ref[0] = _unpatchify_block(recon)
    dz = z_ref[...] - emb
    vq_p = jnp.sum(jnp.sum(dz * dz, axis=1, keepdims=True), axis=0,
                   keepdims=True)
    dr = recon - _patchify_block(x_ref[0])
    mse_p = jnp.sum(jnp.sum(dr * dr, axis=1, keepdims=True), axis=0,
                    keepdims=True)

    @pl.when(pl.program_id(0) == 0)
    def _init():
        vq_ref[...] = vq_p
        mse_ref[...] = mse_p

    @pl.when(pl.program_id(0) != 0)
    def _acc():
        vq_ref[...] += vq_p
        mse_ref[...] += mse_p


@functools.partial(
    pl.kernel,
    mesh=plsc.VectorSubcoreMesh(core_axis_name="c", subcore_axis_name="s"),
    out_type=jax.ShapeDtypeStruct((_TOK_TOTAL, DP), jnp.float32),
    scratch_types=[
        pltpu.VMEM((_NCHUNK, _CHUNK), jnp.int32),
        pltpu.VMEM((_BPW, DP), jnp.float32),
        pltpu.SemaphoreType.DMA,
    ],
)
def _sc_gather(cb_hbm, idx_hbm, out_hbm, idx_v, rows_v, sem):
    wid = lax.axis_index("s") * _NC + lax.axis_index("c")
    base = wid * _BPW
    pltpu.sync_copy(idx_hbm.at[wid], idx_v)
    for j in range(_NCHUNK):
        pltpu.async_copy(cb_hbm.at[idx_v.at[j]],
                         rows_v.at[pl.ds(j * _CHUNK, _CHUNK)], sem).wait()
    pltpu.sync_copy(rows_v, out_hbm.at[pl.ds(base, _BPW)])


def kernel(inputs, W_enc, b_enc, codebook, W_dec, b_dec):
    Bb, Cc, H, W = inputs.shape
    h, w = H // P, W // P
    T = Bb * h * w
    pd = Cc * P * P
    cn_half = 0.5 * jnp.sum(codebook * codebook, axis=1)[None, :]
    pad = DP - D
    W_enc_p = jnp.pad(W_enc, ((0, 0), (0, pad)))
    b_enc_p = jnp.pad(b_enc.reshape(1, D), ((0, 0), (0, pad)))
    cb_p = jnp.pad(codebook, ((0, 0), (0, pad)))
    W_dec_p = jnp.pad(W_dec, ((0, pad), (0, 0)))
    ntok = h * w

    z, idx3 = pl.pallas_call(
        _encode_argmin_body,
        grid=(Bb,),
        in_specs=[
            pl.BlockSpec((1, Cc, H, W), lambda i: (i, 0, 0, 0)),
            pl.BlockSpec((pd, DP), lambda i: (0, 0)),
            pl.BlockSpec((1, DP), lambda i: (0, 0)),
            pl.BlockSpec((K, DP), lambda i: (0, 0)),
            pl.BlockSpec((1, K), lambda i: (0, 0)),
        ],
        out_specs=[
            pl.BlockSpec((ntok, DP), lambda i: (i, 0)),
            pl.BlockSpec((1, 1, ntok), lambda i: (i, 0, 0)),
        ],
        out_shape=[
            jax.ShapeDtypeStruct((T, DP), jnp.float32),
            jax.ShapeDtypeStruct((Bb, 1, ntok), jnp.int32),
        ],
    )(inputs, W_enc_p, b_enc_p, cb_p, cn_half)

    idx = idx3.reshape(_NW, _NCHUNK, _CHUNK)
    emb = _sc_gather(cb_p, idx)

    recon, vq_sse, mse_sse = pl.pallas_call(
        _decode_loss_body,
        grid=(Bb,),
        in_specs=[
            pl.BlockSpec((1, Cc, H, W), lambda i: (i, 0, 0, 0)),
            pl.BlockSpec((ntok, DP), lambda i: (i, 0)),
            pl.BlockSpec((ntok, DP), lambda i: (i, 0)),
            pl.BlockSpec((DP, pd), lambda i: (0, 0)),
            pl.BlockSpec((1, pd), lambda i: (0, 0)),
        ],
        out_specs=[
            pl.BlockSpec((1, Cc, H, W), lambda i: (i, 0, 0, 0)),
            pl.BlockSpec((1, 1), lambda i: (0, 0)),
            pl.BlockSpec((1, 1), lambda i: (0, 0)),
        ],
        out_shape=[
            jax.ShapeDtypeStruct((Bb, Cc, H, W), jnp.float32),
            jax.ShapeDtypeStruct((1, 1), jnp.float32),
            jax.ShapeDtypeStruct((1, 1), jnp.float32),
        ],
    )(inputs, z, emb, W_dec_p, b_dec.reshape(1, pd))

    mse = mse_sse[0, 0] / (Bb * Cc * H * W)
    vq = vq_sse[0, 0] / (T * D)
    loss = 1.25 * vq + mse
    return (loss, mse, recon)
